# baseline (speedup 1.0000x reference)
import jax
import jax.numpy as jnp
from jax.experimental import pallas as pl
from jax.experimental.pallas import tpu as pltpu

_EPS = 1e-5
_B, _C, _H, _W = 32, 3, 400, 600
_KS = 5
_HO, _WO, _OC = _H // _KS, _W // _KS, 16
_HT = 8
_RT = _HT * _KS
_NHT = _HO // _HT
_G = 2


def _fused_kernel(x_ref, wr_ref, s_ref, b_ref, whz_ref, o_ref):
    def tile_body(i, carry):
        a0, a1 = carry
        for sub in range(_G):
            ht = i * _G + sub
            r0 = ht * _RT
            h0 = ht * _HT
            xb = jnp.concatenate(
                [x_ref[0, c, pl.ds(r0, _RT), :].astype(jnp.bfloat16)
                 for c in range(_C)], axis=0)
            for oc in range(_OC):
                t = xb * wr_ref[oc]
                rows = jnp.dot(s_ref[...], t,
                               preferred_element_type=jnp.float32)
                p2 = rows + pltpu.roll(rows, _W - 1, 1)
                p4 = p2 + pltpu.roll(p2, _W - 2, 1)
                s = p4 + pltpu.roll(rows, _W - 4, 1)
                r = jnp.maximum(s + b_ref[oc], 0.0)
                a0 = a0 + jnp.sum(r * whz_ref[0, oc, pl.ds(h0, _HT), :],
                                  axis=0, keepdims=True)
                a1 = a1 + jnp.sum(r * whz_ref[1, oc, pl.ds(h0, _HT), :],
                                  axis=0, keepdims=True)
        return (a0, a1)

    zero = jnp.zeros((1, _W), jnp.float32)
    a0, a1 = jax.lax.fori_loop(0, _NHT // _G, tile_body, (zero, zero))
    t0 = jnp.sum(a0)
    t1 = jnp.sum(a1)
    lane = jax.lax.broadcasted_iota(jnp.int32, (1, 1, 128), 2)
    o_ref[...] = jnp.where(lane == 0, t0, jnp.where(lane == 1, t1, 0.0))


def kernel(x, conv_w, conv_b, bn_gamma, bn_beta, bn_mean, bn_var,
           head_w, head_b):
    bn_scale = bn_gamma * jax.lax.rsqrt(bn_var + _EPS)
    w_sc = conv_w * bn_scale[:, None, None, None]
    b_eff = bn_scale * (conv_b - bn_mean) + bn_beta

    wr = jnp.tile(w_sc, (1, 1, _RT // _KS, _WO)).astype(jnp.bfloat16)
    wr = wr.reshape(_OC, _C * _RT, _W)

    row = jax.lax.broadcasted_iota(jnp.int32, (_HT, _RT), 0)
    col = jax.lax.broadcasted_iota(jnp.int32, (_HT, _RT), 1)
    s_band = (col >= _KS * row) & (col < _KS * row + _KS)
    s_mat = jnp.tile(s_band, (1, _C)).astype(jnp.bfloat16)

    wh = head_w.reshape(2, _OC, _HO, _WO)
    whz = jnp.zeros((2, _OC, _HO, _W), jnp.float32)
    whz = whz.at[:, :, :, ::_KS].set(wh)

    out_pad = pl.pallas_call(
        _fused_kernel,
        out_shape=jax.ShapeDtypeStruct((_B, 1, 128), jnp.float32),
        grid_spec=pltpu.PrefetchScalarGridSpec(
            num_scalar_prefetch=0,
            grid=(_B,),
            in_specs=[
                pl.BlockSpec((1, _C, _H, _W), lambda b: (b, 0, 0, 0)),
                pl.BlockSpec((_OC, _C * _RT, _W), lambda b: (0, 0, 0)),
                pl.BlockSpec((_HT, _C * _RT), lambda b: (0, 0)),
                pl.BlockSpec(memory_space=pltpu.SMEM),
                pl.BlockSpec((2, _OC, _HO, _W), lambda b: (0, 0, 0, 0)),
            ],
            out_specs=pl.BlockSpec((1, 1, 128), lambda b: (b, 0, 0)),
        ),
        compiler_params=pltpu.CompilerParams(
            dimension_semantics=("parallel",)),
    )(x, wr, s_mat, b_eff, whz)

    return out_pad[:, 0, :2] + head_b[None, :]


# speedup vs baseline: 1.0065x; 1.0065x over previous
import jax
import jax.numpy as jnp
from jax.experimental import pallas as pl
from jax.experimental.pallas import tpu as pltpu

_EPS = 1e-5
_B, _C, _H, _W = 32, 3, 400, 600
_KS = 5
_HO, _WO, _OC = _H // _KS, _W // _KS, 16
_HT = 8
_RT = _HT * _KS
_NHT = _HO // _HT
_G = 2


def _fused_kernel(x_ref, wr_ref, s_ref, b_ref, whz_ref, o_ref):
    def tile_body(i, carry):
        a0, a1 = carry
        for sub in range(_G):
            ht = i * _G + sub
            r0 = ht * _RT
            h0 = ht * _HT
            xb = [x_ref[0, c, pl.ds(r0, _RT), :].astype(jnp.bfloat16)
                  for c in range(_C)]
            for oc in range(_OC):
                acc = xb[0] * wr_ref[oc, 0]
                for c in range(1, _C):
                    acc = acc + xb[c] * wr_ref[oc, c]
                rows = jnp.dot(s_ref[...], acc,
                               preferred_element_type=jnp.float32)
                p2 = rows + pltpu.roll(rows, _W - 1, 1)
                p4 = p2 + pltpu.roll(p2, _W - 2, 1)
                s = p4 + pltpu.roll(rows, _W - 4, 1)
                r = jnp.maximum(s + b_ref[oc], 0.0)
                a0 = a0 + jnp.sum(r * whz_ref[0, oc, pl.ds(h0, _HT), :],
                                  axis=0, keepdims=True)
                a1 = a1 + jnp.sum(r * whz_ref[1, oc, pl.ds(h0, _HT), :],
                                  axis=0, keepdims=True)
        return (a0, a1)

    zero = jnp.zeros((1, _W), jnp.float32)
    a0, a1 = jax.lax.fori_loop(0, _NHT // _G, tile_body, (zero, zero))
    t0 = jnp.sum(a0)
    t1 = jnp.sum(a1)
    lane = jax.lax.broadcasted_iota(jnp.int32, (1, 1, 128), 2)
    o_ref[...] = jnp.where(lane == 0, t0, jnp.where(lane == 1, t1, 0.0))


def kernel(x, conv_w, conv_b, bn_gamma, bn_beta, bn_mean, bn_var,
           head_w, head_b):
    bn_scale = bn_gamma * jax.lax.rsqrt(bn_var + _EPS)
    w_sc = conv_w * bn_scale[:, None, None, None]
    b_eff = bn_scale * (conv_b - bn_mean) + bn_beta

    wr = jnp.tile(w_sc, (1, 1, _RT // _KS, _WO)).astype(jnp.bfloat16)

    row = jax.lax.broadcasted_iota(jnp.int32, (_HT, _RT), 0)
    col = jax.lax.broadcasted_iota(jnp.int32, (_HT, _RT), 1)
    s_mat = ((col >= _KS * row)
             & (col < _KS * row + _KS)).astype(jnp.bfloat16)

    wh = head_w.reshape(2, _OC, _HO, _WO)
    whz = jnp.zeros((2, _OC, _HO, _W), jnp.float32)
    whz = whz.at[:, :, :, ::_KS].set(wh)

    out_pad = pl.pallas_call(
        _fused_kernel,
        out_shape=jax.ShapeDtypeStruct((_B, 1, 128), jnp.float32),
        grid_spec=pltpu.PrefetchScalarGridSpec(
            num_scalar_prefetch=0,
            grid=(_B,),
            in_specs=[
                pl.BlockSpec((1, _C, _H, _W), lambda b: (b, 0, 0, 0)),
                pl.BlockSpec((_OC, _C, _RT, _W), lambda b: (0, 0, 0, 0)),
                pl.BlockSpec((_HT, _RT), lambda b: (0, 0)),
                pl.BlockSpec(memory_space=pltpu.SMEM),
                pl.BlockSpec((2, _OC, _HO, _W), lambda b: (0, 0, 0, 0)),
            ],
            out_specs=pl.BlockSpec((1, 1, 128), lambda b: (b, 0, 0)),
        ),
        compiler_params=pltpu.CompilerParams(
            dimension_semantics=("parallel",)),
    )(x, wr, s_mat, b_eff, whz)

    return out_pad[:, 0, :2] + head_b[None, :]
